# Initial kernel scaffold; baseline (speedup 1.0000x reference)
import jax
import jax.numpy as jnp
from jax.experimental import pallas as pl


def kernel(mem, idx, val, lower, upper):
    raise NotImplementedError("write your pallas kernel here")


# speedup vs baseline: 2.0592x; 2.0592x over previous
import functools

import jax
import jax.numpy as jnp
from jax import lax
from jax.experimental import pallas as pl
from jax.experimental.pallas import tpu as pltpu
from jax.experimental.pallas import tpu_sc as plsc

_LANES = 16
_CH = 128


def _make_scatter_kernel(M, D, B, NC, NS, interpret=False):
    NW = NC * NS
    b_per_w = B // NW
    nch = b_per_w // _CH
    mesh = plsc.VectorSubcoreMesh(core_axis_name="c", subcore_axis_name="s")

    def body(out_hbm, idx_hbm, val_hbm, lo_hbm, up_hbm,
             idx_v, rows_v, lo_v, up_v, sem):
        wid = lax.axis_index("s") * NC + lax.axis_index("c")
        pltpu.sync_copy(idx_hbm.at[wid], idx_v)
        pltpu.sync_copy(val_hbm.at[wid], rows_v)
        pltpu.sync_copy(lo_hbm, lo_v)
        pltpu.sync_copy(up_hbm, up_v)

        nvec = D // _LANES
        lo = [lo_v[pl.ds(k * _LANES, _LANES)] for k in range(nvec)]
        up = [up_v[pl.ds(k * _LANES, _LANES)] for k in range(nvec)]

        def row(i, _):
            for k in range(nvec):
                v = rows_v[i, pl.ds(k * _LANES, _LANES)]
                d = jnp.where(v >= up[k], lo[k],
                              jnp.where(v <= lo[k], up[k], v))
                rows_v[i, pl.ds(k * _LANES, _LANES)] = d
            return 0

        lax.fori_loop(0, b_per_w, row, 0)

        for j in range(nch):
            pltpu.async_copy(rows_v.at[pl.ds(j * _CH, _CH)],
                             out_hbm.at[idx_v.at[j]], sem).wait()

    return pl.kernel(
        body,
        out_type=(),
        mesh=mesh,
        scratch_types=[
            pltpu.VMEM((nch, _CH), jnp.int32),
            pltpu.VMEM((b_per_w, D), jnp.float32),
            pltpu.VMEM((D,), jnp.float32),
            pltpu.VMEM((D,), jnp.float32),
            pltpu.SemaphoreType.DMA,
        ],
        compiler_params=pltpu.CompilerParams(use_tc_tiling_on_sc=False),
        interpret=interpret,
    )


def kernel(mem, idx, val, lower, upper):
    M, D = mem.shape
    B = idx.shape[0]
    NC, NS = 2, 16
    NW = NC * NS
    b_per_w = B // NW
    nch = b_per_w // _CH

    idx3 = idx.reshape(NW, nch, _CH)
    val3 = val.reshape(NW, b_per_w, D)

    scatter = _make_scatter_kernel(M, D, B, NC, NS)
    mem_ref = jax.new_ref(mem)
    scatter(mem_ref, idx3, val3, lower, upper)
    return mem_ref[...]
